# baseline (speedup 1.0000x reference)
import functools

import jax
import jax.numpy as jnp
from jax import lax
from jax.experimental import pallas as pl
from jax.experimental.pallas import tpu as pltpu
from jax.experimental.pallas import tpu_sc as plsc

N_NODES = 100000
D_FEAT = 128
HIDDEN = 64
HP = HIDDEN // 2
S = 5
BATCH = 16384

NC = 2
NS = 16
L = 16
NW = NC * NS

_MASK_HI = -65536
_MASK_LO = 65535
_RND = 32768


def _i32(x, v):
    return jnp.full(x.shape, v, jnp.int32)


def _pack_pair_i32(lo_f32, hi_f32):
    il = lax.bitcast_convert_type(lo_f32, jnp.int32)
    ih = lax.bitcast_convert_type(hi_f32, jnp.int32)
    wl = lax.shift_right_arithmetic(il + _i32(il, _RND), _i32(il, 16))
    wl = wl & _i32(wl, _MASK_LO)
    wh = (ih + _i32(ih, _RND)) & _i32(ih, _MASK_HI)
    return wl | wh


def _unpack_pair_f32(w_i32):
    lo = lax.bitcast_convert_type(
        lax.shift_left(w_i32, _i32(w_i32, 16)), jnp.float32)
    hi = lax.bitcast_convert_type(w_i32 & _i32(w_i32, _MASK_HI), jnp.float32)
    return lo, hi


_PROJ_BLK = 2000


def _proj_body(x_ref, wa_ref, wb_ref, pa_ref, pb_ref):
    x = x_ref[...]
    dn = (((1,), (1,)), ((), ()))
    for w_ref, o_ref in ((wa_ref, pa_ref), (wb_ref, pb_ref)):
        p = lax.dot_general(x, w_ref[...], dn,
                            preferred_element_type=jnp.float32)
        w = _pack_pair_i32(p[:, :HP], p[:, HP:])
        o_ref[...] = lax.bitcast_convert_type(w, jnp.float32)


def _project(features, w1a, w1b):
    grid = (N_NODES // _PROJ_BLK,)
    return pl.pallas_call(
        _proj_body,
        grid=grid,
        in_specs=[
            pl.BlockSpec((_PROJ_BLK, D_FEAT), lambda i: (i, 0)),
            pl.BlockSpec((HIDDEN, D_FEAT), lambda i: (0, 0)),
            pl.BlockSpec((HIDDEN, D_FEAT), lambda i: (0, 0)),
        ],
        out_specs=[
            pl.BlockSpec((_PROJ_BLK, HP), lambda i: (i, 0)),
            pl.BlockSpec((_PROJ_BLK, HP), lambda i: (i, 0)),
        ],
        out_shape=[jax.ShapeDtypeStruct((N_NODES, HP), jnp.float32)] * 2,
    )(features, w1a, w1b)


_NB = 8
_CH1 = 160
_NCH1 = N_NODES // _CH1
_ITERS1 = -(-_NCH1 // NW)

_sc_mesh = plsc.VectorSubcoreMesh(core_axis_name="c", subcore_axis_name="s")
_sc_params = pltpu.CompilerParams(use_tc_tiling_on_sc=False)


@functools.partial(
    pl.kernel,
    out_type=jax.ShapeDtypeStruct((N_NODES, HP), jnp.float32),
    mesh=_sc_mesh,
    compiler_params=_sc_params,
    scratch_types=(
        [pltpu.VMEM((_CH1,), jnp.int32) for _ in range(S)]
        + [pltpu.VMEM((_CH1, HP), jnp.float32) for _ in range(S)]
        + [
            pltpu.VMEM((_CH1, HP), jnp.float32),
            pltpu.VMEM((_CH1, HP), jnp.float32),
            pltpu.SemaphoreType.DMA,
        ]
    ),
)
def _agg1(pa_hbm, pb_hbm, n0, n1, n2, n3, n4, h1_hbm,
          i0, i1, i2, i3, i4, r0, r1, r2, r3, r4, pa_v, out_v, sem):
    ncols = (n0, n1, n2, n3, n4)
    idx_v = (i0, i1, i2, i3, i4)
    rows_v = (r0, r1, r2, r3, r4)
    wid = lax.axis_index("s") * NC + lax.axis_index("c")

    def chunk_body(i, carry):
        k = i * NW + wid

        @pl.when(k < _NCH1)
        def _():
            base = k * _CH1
            for s in range(S):
                pltpu.sync_copy(ncols[s].at[pl.ds(base, _CH1)], idx_v[s])
            pltpu.sync_copy(pa_hbm.at[pl.ds(base, _CH1)], pa_v)
            cps = [pltpu.async_copy(pb_hbm.at[idx_v[s]], rows_v[s], sem)
                   for s in range(S)]
            for cp in cps:
                cp.wait()

            def node_body(c, carry2):
                blk = pl.ds(c * _NB, _NB)
                slo = shi = None
                for s in range(S):
                    w = lax.bitcast_convert_type(rows_v[s][blk, :],
                                                 jnp.int32)
                    lo, hi = _unpack_pair_f32(w)
                    slo = lo if slo is None else slo + lo
                    shi = hi if shi is None else shi + hi
                pw = lax.bitcast_convert_type(pa_v[blk, :], jnp.int32)
                plo, phi = _unpack_pair_f32(pw)
                h_lo = jnp.maximum(plo + 0.2 * slo, 0.0)
                h_hi = jnp.maximum(phi + 0.2 * shi, 0.0)
                out_v[blk, :] = lax.bitcast_convert_type(
                    _pack_pair_i32(h_lo, h_hi), jnp.float32)
                return carry2

            lax.fori_loop(0, _CH1 // _NB, node_body, 0)
            pltpu.sync_copy(out_v, h1_hbm.at[pl.ds(base, _CH1)])

        return carry

    lax.fori_loop(0, _ITERS1, chunk_body, 0)


_CH2 = 256
_ITERS2 = BATCH // (NW * _CH2)


@functools.partial(
    pl.kernel,
    out_type=[jax.ShapeDtypeStruct((BATCH, HP), jnp.float32)] * 2,
    mesh=_sc_mesh,
    compiler_params=_sc_params,
    scratch_types=(
        [pltpu.VMEM((_CH2,), jnp.int32)]
        + [pltpu.VMEM((_CH2,), jnp.int32) for _ in range(S)]
        + [pltpu.VMEM((_CH2, HP), jnp.float32) for _ in range(S)]
        + [
            pltpu.VMEM((_CH2, HP), jnp.float32),
            pltpu.VMEM((_CH2, HP), jnp.float32),
            pltpu.SemaphoreType.DMA,
        ]
    ),
)
def _agg2(nodes_hbm, h1_hbm, n0, n1, n2, n3, n4, gs_hbm, gn_hbm,
          nd_v, m0, m1, m2, m3, m4, r0, r1, r2, r3, r4, self_v, gn_v, sem):
    ncols = (n0, n1, n2, n3, n4)
    n2_v = (m0, m1, m2, m3, m4)
    rows_v = (r0, r1, r2, r3, r4)
    wid = lax.axis_index("s") * NC + lax.axis_index("c")

    def chunk_body(i, carry):
        k = i * NW + wid
        base = k * _CH2
        pltpu.sync_copy(nodes_hbm.at[pl.ds(base, _CH2)], nd_v)
        cps = [pltpu.async_copy(ncols[s].at[nd_v], n2_v[s], sem)
               for s in range(S)]
        cps.append(pltpu.async_copy(h1_hbm.at[nd_v], self_v, sem))
        for cp in cps:
            cp.wait()
        cps = [pltpu.async_copy(h1_hbm.at[n2_v[s]], rows_v[s], sem)
               for s in range(S)]
        for cp in cps:
            cp.wait()
        pltpu.sync_copy(self_v, gs_hbm.at[pl.ds(base, _CH2)])

        def node_body(c, carry2):
            blk = pl.ds(c * _NB, _NB)
            slo = shi = None
            for s in range(S):
                w = lax.bitcast_convert_type(rows_v[s][blk, :], jnp.int32)
                lo, hi = _unpack_pair_f32(w)
                slo = lo if slo is None else slo + lo
                shi = hi if shi is None else shi + hi
            gn_v[blk, :] = lax.bitcast_convert_type(
                _pack_pair_i32(0.2 * slo, 0.2 * shi), jnp.float32)
            return carry2

        lax.fori_loop(0, _CH2 // _NB, node_body, 0)
        pltpu.sync_copy(gn_v, gn_hbm.at[pl.ds(base, _CH2)])
        return carry

    lax.fori_loop(0, _ITERS2, chunk_body, 0)


_HEAD_BLK = 2048


def _head_body(gs_ref, gn_ref, w2al_ref, w2ah_ref, w2bl_ref, w2bh_ref,
               wc_ref, out_ref):
    dn = (((1,), (1,)), ((), ()))
    acc = None
    for g_ref, wl_ref, wh_ref in ((gs_ref, w2al_ref, w2ah_ref),
                                  (gn_ref, w2bl_ref, w2bh_ref)):
        w = lax.bitcast_convert_type(g_ref[...], jnp.int32)
        lo, hi = _unpack_pair_f32(w)
        part = lax.dot_general(lo, wl_ref[...], dn,
                               preferred_element_type=jnp.float32)
        part = part + lax.dot_general(hi, wh_ref[...], dn,
                                      preferred_element_type=jnp.float32)
        acc = part if acc is None else acc + part
    h = jnp.maximum(acc, 0.0)
    out_ref[...] = lax.dot_general(h, wc_ref[...], dn,
                                   preferred_element_type=jnp.float32)


def _head(gs, gn, w2al, w2ah, w2bl, w2bh, wc, num_classes):
    grid = (BATCH // _HEAD_BLK,)
    wspec = pl.BlockSpec((HIDDEN, HP), lambda i: (0, 0))
    return pl.pallas_call(
        _head_body,
        grid=grid,
        in_specs=[
            pl.BlockSpec((_HEAD_BLK, HP), lambda i: (i, 0)),
            pl.BlockSpec((_HEAD_BLK, HP), lambda i: (i, 0)),
            wspec, wspec, wspec, wspec,
            pl.BlockSpec((num_classes, HIDDEN), lambda i: (0, 0)),
        ],
        out_specs=pl.BlockSpec((_HEAD_BLK, num_classes), lambda i: (i, 0)),
        out_shape=jax.ShapeDtypeStruct((BATCH, num_classes), jnp.float32),
    )(gs, gn, w2al, w2ah, w2bl, w2bh, wc)


def kernel(nodes, features, neigh_idx, W1, W2, Wc):
    w1a = W1[:, :D_FEAT]
    w1b = W1[:, D_FEAT:]
    w2a = W2[:, :HIDDEN]
    w2b = W2[:, HIDDEN:]
    w2al, w2ah = w2a[:, :HP], w2a[:, HP:]
    w2bl, w2bh = w2b[:, :HP], w2b[:, HP:]
    ncols = [neigh_idx[:, s] for s in range(S)]

    pa, pb = _project(features, w1a, w1b)
    h1 = _agg1(pa, pb, *ncols)
    gs, gn = _agg2(nodes, h1, *ncols)
    return _head(gs, gn, w2al, w2ah, w2bl, w2bh, Wc, Wc.shape[0])


# speedup vs baseline: 1.0114x; 1.0114x over previous
import functools

import jax
import jax.numpy as jnp
from jax import lax
from jax.experimental import pallas as pl
from jax.experimental.pallas import tpu as pltpu
from jax.experimental.pallas import tpu_sc as plsc

N_NODES = 100000
D_FEAT = 128
HIDDEN = 64
HP = HIDDEN // 2
S = 5
BATCH = 16384

NC = 2
NS = 16
L = 16
NW = NC * NS

_MASK_HI = -65536
_MASK_LO = 65535
_RND = 32768


def _i32(x, v):
    return jnp.full(x.shape, v, jnp.int32)


def _pack_pair_i32(lo_f32, hi_f32):
    il = lax.bitcast_convert_type(lo_f32, jnp.int32)
    ih = lax.bitcast_convert_type(hi_f32, jnp.int32)
    wl = lax.shift_right_arithmetic(il + _i32(il, _RND), _i32(il, 16))
    wl = wl & _i32(wl, _MASK_LO)
    wh = (ih + _i32(ih, _RND)) & _i32(ih, _MASK_HI)
    return wl | wh


def _unpack_pair_f32(w_i32):
    lo = lax.bitcast_convert_type(
        lax.shift_left(w_i32, _i32(w_i32, 16)), jnp.float32)
    hi = lax.bitcast_convert_type(w_i32 & _i32(w_i32, _MASK_HI), jnp.float32)
    return lo, hi


_PROJ_BLK = 2000


def _proj_body(x_ref, wa_ref, wb_ref, pa_ref, pb_ref):
    x = x_ref[...]
    dn = (((1,), (1,)), ((), ()))
    for w_ref, o_ref in ((wa_ref, pa_ref), (wb_ref, pb_ref)):
        p = lax.dot_general(x, w_ref[...], dn,
                            preferred_element_type=jnp.float32)
        w = _pack_pair_i32(p[:, :HP], p[:, HP:])
        o_ref[...] = lax.bitcast_convert_type(w, jnp.float32)


def _project(features, w1a, w1b):
    grid = (N_NODES // _PROJ_BLK,)
    return pl.pallas_call(
        _proj_body,
        grid=grid,
        in_specs=[
            pl.BlockSpec((_PROJ_BLK, D_FEAT), lambda i: (i, 0)),
            pl.BlockSpec((HIDDEN, D_FEAT), lambda i: (0, 0)),
            pl.BlockSpec((HIDDEN, D_FEAT), lambda i: (0, 0)),
        ],
        out_specs=[
            pl.BlockSpec((_PROJ_BLK, HP), lambda i: (i, 0)),
            pl.BlockSpec((_PROJ_BLK, HP), lambda i: (i, 0)),
        ],
        out_shape=[jax.ShapeDtypeStruct((N_NODES, HP), jnp.float32)] * 2,
    )(features, w1a, w1b)


_CH1 = 160
_NCH1 = N_NODES // _CH1
_ITERS1 = -(-_NCH1 // NW)

_sc_mesh = plsc.VectorSubcoreMesh(core_axis_name="c", subcore_axis_name="s")
_sc_params = pltpu.CompilerParams(use_tc_tiling_on_sc=False)


@functools.partial(
    pl.kernel,
    out_type=jax.ShapeDtypeStruct((N_NODES, HP), jnp.float32),
    mesh=_sc_mesh,
    compiler_params=_sc_params,
    scratch_types=(
        [pltpu.VMEM((_CH1,), jnp.int32) for _ in range(S)]
        + [pltpu.VMEM((_CH1, HP), jnp.float32) for _ in range(S)]
        + [
            pltpu.VMEM((_CH1, HP), jnp.float32),
            pltpu.VMEM((_CH1, HP), jnp.float32),
            pltpu.SemaphoreType.DMA,
        ]
    ),
)
def _agg1(pa_hbm, pb_hbm, n0, n1, n2, n3, n4, h1_hbm,
          i0, i1, i2, i3, i4, r0, r1, r2, r3, r4, pa_v, out_v, sem):
    ncols = (n0, n1, n2, n3, n4)
    idx_v = (i0, i1, i2, i3, i4)
    rows_v = (r0, r1, r2, r3, r4)
    wid = lax.axis_index("s") * NC + lax.axis_index("c")

    def chunk_body(i, carry):
        k = i * NW + wid

        @pl.when(k < _NCH1)
        def _():
            base = k * _CH1
            for s in range(S):
                pltpu.sync_copy(ncols[s].at[pl.ds(base, _CH1)], idx_v[s])
            pltpu.sync_copy(pa_hbm.at[pl.ds(base, _CH1)], pa_v)
            cps = [pltpu.async_copy(pb_hbm.at[idx_v[s]], rows_v[s], sem)
                   for s in range(S)]
            for cp in cps:
                cp.wait()

            def node_body(c, carry2):
                for g in range(HP // L):
                    sl = pl.ds(g * L, L)
                    slo = shi = None
                    for s in range(S):
                        w = lax.bitcast_convert_type(rows_v[s][c, sl],
                                                     jnp.int32)
                        lo, hi = _unpack_pair_f32(w)
                        slo = lo if slo is None else slo + lo
                        shi = hi if shi is None else shi + hi
                    pw = lax.bitcast_convert_type(pa_v[c, sl], jnp.int32)
                    plo, phi = _unpack_pair_f32(pw)
                    h_lo = jnp.maximum(plo + 0.2 * slo, 0.0)
                    h_hi = jnp.maximum(phi + 0.2 * shi, 0.0)
                    out_v[c, sl] = lax.bitcast_convert_type(
                        _pack_pair_i32(h_lo, h_hi), jnp.float32)
                return carry2

            lax.fori_loop(0, _CH1, node_body, 0)
            pltpu.sync_copy(out_v, h1_hbm.at[pl.ds(base, _CH1)])

        return carry

    lax.fori_loop(0, _ITERS1, chunk_body, 0)


_CH2 = 256
_ITERS2 = BATCH // (NW * _CH2)


@functools.partial(
    pl.kernel,
    out_type=[jax.ShapeDtypeStruct((BATCH, HP), jnp.float32)] * 2,
    mesh=_sc_mesh,
    compiler_params=_sc_params,
    scratch_types=(
        [pltpu.VMEM((_CH2,), jnp.int32)]
        + [pltpu.VMEM((_CH2,), jnp.int32) for _ in range(S)]
        + [pltpu.VMEM((_CH2, HP), jnp.float32) for _ in range(S)]
        + [
            pltpu.VMEM((_CH2, HP), jnp.float32),
            pltpu.VMEM((_CH2, HP), jnp.float32),
            pltpu.SemaphoreType.DMA,
        ]
    ),
)
def _agg2(nodes_hbm, h1_hbm, n0, n1, n2, n3, n4, gs_hbm, gn_hbm,
          nd_v, m0, m1, m2, m3, m4, r0, r1, r2, r3, r4, self_v, gn_v, sem):
    ncols = (n0, n1, n2, n3, n4)
    n2_v = (m0, m1, m2, m3, m4)
    rows_v = (r0, r1, r2, r3, r4)
    wid = lax.axis_index("s") * NC + lax.axis_index("c")

    def chunk_body(i, carry):
        k = i * NW + wid
        base = k * _CH2
        pltpu.sync_copy(nodes_hbm.at[pl.ds(base, _CH2)], nd_v)
        cps = [pltpu.async_copy(ncols[s].at[nd_v], n2_v[s], sem)
               for s in range(S)]
        cps.append(pltpu.async_copy(h1_hbm.at[nd_v], self_v, sem))
        for cp in cps:
            cp.wait()
        cps = [pltpu.async_copy(h1_hbm.at[n2_v[s]], rows_v[s], sem)
               for s in range(S)]
        for cp in cps:
            cp.wait()
        pltpu.sync_copy(self_v, gs_hbm.at[pl.ds(base, _CH2)])

        def node_body(c, carry2):
            for g in range(HP // L):
                sl = pl.ds(g * L, L)
                slo = shi = None
                for s in range(S):
                    w = lax.bitcast_convert_type(rows_v[s][c, sl], jnp.int32)
                    lo, hi = _unpack_pair_f32(w)
                    slo = lo if slo is None else slo + lo
                    shi = hi if shi is None else shi + hi
                gn_v[c, sl] = lax.bitcast_convert_type(
                    _pack_pair_i32(0.2 * slo, 0.2 * shi), jnp.float32)
            return carry2

        lax.fori_loop(0, _CH2, node_body, 0)
        pltpu.sync_copy(gn_v, gn_hbm.at[pl.ds(base, _CH2)])
        return carry

    lax.fori_loop(0, _ITERS2, chunk_body, 0)


_HEAD_BLK = 2048


def _head_body(gs_ref, gn_ref, w2al_ref, w2ah_ref, w2bl_ref, w2bh_ref,
               wc_ref, out_ref):
    dn = (((1,), (1,)), ((), ()))
    acc = None
    for g_ref, wl_ref, wh_ref in ((gs_ref, w2al_ref, w2ah_ref),
                                  (gn_ref, w2bl_ref, w2bh_ref)):
        w = lax.bitcast_convert_type(g_ref[...], jnp.int32)
        lo, hi = _unpack_pair_f32(w)
        part = lax.dot_general(lo, wl_ref[...], dn,
                               preferred_element_type=jnp.float32)
        part = part + lax.dot_general(hi, wh_ref[...], dn,
                                      preferred_element_type=jnp.float32)
        acc = part if acc is None else acc + part
    h = jnp.maximum(acc, 0.0)
    out_ref[...] = lax.dot_general(h, wc_ref[...], dn,
                                   preferred_element_type=jnp.float32)


def _head(gs, gn, w2al, w2ah, w2bl, w2bh, wc, num_classes):
    grid = (BATCH // _HEAD_BLK,)
    wspec = pl.BlockSpec((HIDDEN, HP), lambda i: (0, 0))
    return pl.pallas_call(
        _head_body,
        grid=grid,
        in_specs=[
            pl.BlockSpec((_HEAD_BLK, HP), lambda i: (i, 0)),
            pl.BlockSpec((_HEAD_BLK, HP), lambda i: (i, 0)),
            wspec, wspec, wspec, wspec,
            pl.BlockSpec((num_classes, HIDDEN), lambda i: (0, 0)),
        ],
        out_specs=pl.BlockSpec((_HEAD_BLK, num_classes), lambda i: (i, 0)),
        out_shape=jax.ShapeDtypeStruct((BATCH, num_classes), jnp.float32),
    )(gs, gn, w2al, w2ah, w2bl, w2bh, wc)


def kernel(nodes, features, neigh_idx, W1, W2, Wc):
    w1a = W1[:, :D_FEAT]
    w1b = W1[:, D_FEAT:]
    w2a = W2[:, :HIDDEN]
    w2b = W2[:, HIDDEN:]
    w2al, w2ah = w2a[:, :HP], w2a[:, HP:]
    w2bl, w2bh = w2b[:, :HP], w2b[:, HP:]
    ncols = [neigh_idx[:, s] for s in range(S)]

    pa, pb = _project(features, w1a, w1b)
    h1 = _agg1(pa, pb, *ncols)
    gs, gn = _agg2(nodes, h1, *ncols)
    return _head(gs, gn, w2al, w2ah, w2bl, w2bh, Wc, Wc.shape[0])
